# baseline (speedup 1.0000x reference)
import functools

import jax
import jax.numpy as jnp
from jax import lax
from jax.experimental import pallas as pl
from jax.experimental.pallas import tpu as pltpu
from jax.experimental.pallas import tpu_sc as plsc

N = 10000
DEG = 32
E = 128
F = 256
H1 = 400
B = 4096
NSTEP = 3

NC = 2
NS = 16
NW = NC * NS
NPAD = ((N + NW * 8 - 1) // (NW * 8)) * (NW * 8)
NPW = NPAD // NW
RTOT = NSTEP * B
RPW = RTOT // NW
LTOT = (NSTEP - 1) * B
LPW = LTOT // NW
EW = E // 2
NWIN = EW // 16

_mesh = lambda: plsc.VectorSubcoreMesh(core_axis_name="c", subcore_axis_name="s")

_HI = -65536


def _to_f32_pair(w):
  lo = lax.bitcast_convert_type(w << 16, jnp.float32)
  hi = lax.bitcast_convert_type(w & _HI, jnp.float32)
  return lo, hi


def _from_f32_pair(lo, hi):
  bl = lax.shift_right_logical(
      lax.bitcast_convert_type(lo, jnp.int32) + 0x8000, 16)
  bh = (lax.bitcast_convert_type(hi, jnp.int32) + 0x8000) & _HI
  return bl | bh


def _sc_neighbor_mean(emb_w, nidx_flat):
  G4 = 4
  CHI = G4 * DEG
  NGRP = NPW // G4
  NBUF = 2

  SEG = (N + NS - 1) // NS

  @functools.partial(
      pl.kernel,
      out_type=jax.ShapeDtypeStruct((NPAD, EW), jnp.int32),
      mesh=_mesh(),
      scratch_types=[
          pltpu.VMEM((NPW * DEG,), jnp.int32),
          pltpu.VMEM((NBUF, CHI, EW), jnp.int32),
          pltpu.VMEM((NPW, EW), jnp.int32),
          pltpu.VMEM_SHARED((N, EW), jnp.int32),
          pltpu.SemaphoreType.DMA,
          pltpu.SemaphoreType.DMA,
      ],
      compiler_params=pltpu.CompilerParams(use_tc_tiling_on_sc=False),
  )
  def k(emb_hbm, idx_hbm, a_hbm, idx_v, rows_v, out_v, emb_sp, sem0, sem1):
    sems = (sem0, sem1)
    wid = lax.axis_index("s") * NC + lax.axis_index("c")
    base = wid * NPW
    sid = lax.axis_index("s")
    seg0 = sid * SEG
    pltpu.sync_copy(idx_hbm.at[pl.ds(base * DEG, NPW * DEG)], idx_v)
    pltpu.sync_copy(emb_hbm.at[pl.ds(seg0, SEG)], emb_sp.at[pl.ds(seg0, SEG)])
    plsc.subcore_barrier()

    def src_tab(b):
      return emb_sp if b % 2 == 0 else emb_hbm

    def gather(g, b):
      return pltpu.async_copy(
          src_tab(b).at[idx_v.at[pl.ds(g * CHI, CHI)]], rows_v.at[b], sems[b])

    for b in range(NBUF):
      gather(b, b)

    def grp_body(it, carry):
      for b in range(NBUF):
        g = it * NBUF + b
        pltpu.make_async_copy(
            src_tab(b).at[idx_v.at[pl.ds(g * CHI, CHI)]], rows_v.at[b],
            sems[b]).wait()
        rb = rows_v.at[b]
        for t in range(G4):
          for q in range(NWIN):
            acc_lo = None
            acc_hi = None
            for kk in range(DEG):
              lo, hi = _to_f32_pair(rb[t * DEG + kk, pl.ds(q * 16, 16)])
              acc_lo = lo if acc_lo is None else acc_lo + lo
              acc_hi = hi if acc_hi is None else acc_hi + hi
            out_v[g * G4 + t, pl.ds(q * 16, 16)] = _from_f32_pair(
                acc_lo * (1.0 / DEG), acc_hi * (1.0 / DEG))

        @pl.when(g + NBUF < NGRP)
        def _():
          gather(g + NBUF, b)

      return carry

    lax.fori_loop(0, NGRP // NBUF, grp_body, 0)
    pltpu.sync_copy(out_v, a_hbm.at[pl.ds(base, NPW)])

  return k(emb_w, nidx_flat)


def _sc_path_gather(node_emb_w, a_tab, rel_emb_w, root_idx, rel_idx):
  CH = 128
  NB = 4

  @functools.partial(
      pl.kernel,
      out_type=(
          jax.ShapeDtypeStruct((RTOT, EW), jnp.int32),
          jax.ShapeDtypeStruct((RTOT, EW), jnp.int32),
          jax.ShapeDtypeStruct((LTOT, EW), jnp.int32),
      ),
      mesh=_mesh(),
      scratch_types=[
          pltpu.VMEM((RPW,), jnp.int32),
          pltpu.VMEM((LPW,), jnp.int32),
          pltpu.VMEM((NB, CH, EW), jnp.int32),
      ] + [pltpu.SemaphoreType.DMA] * (2 * NB),
      compiler_params=pltpu.CompilerParams(use_tc_tiling_on_sc=False),
  )
  def k(emb_hbm, a_hbm, rel_hbm, ridx_hbm, lidx_hbm, g0_hbm, g1_hbm, g2_hbm,
        ridx_v, lidx_v, rows_v, *sems):
    semg, sems_ = sems[:NB], sems[NB:]
    wid = lax.axis_index("s") * NC + lax.axis_index("c")
    rbase = wid * RPW
    lbase = wid * LPW
    pltpu.sync_copy(ridx_hbm.at[pl.ds(rbase, RPW)], ridx_v)
    pltpu.sync_copy(lidx_hbm.at[pl.ds(lbase, LPW)], lidx_v)
    chunks = (
        [(emb_hbm, ridx_v, ch * CH, g0_hbm, rbase + ch * CH)
         for ch in range(RPW // CH)]
        + [(a_hbm, ridx_v, ch * CH, g1_hbm, rbase + ch * CH)
           for ch in range(RPW // CH)]
        + [(rel_hbm, lidx_v, ch * CH, g2_hbm, lbase + ch * CH)
           for ch in range(LPW // CH)])
    NCH = len(chunks)

    def g_copy(i, b):
      tab, idxr, ioff, _, _ = chunks[i]
      return pltpu.make_async_copy(
          tab.at[idxr.at[pl.ds(ioff, CH)]], rows_v.at[b], semg[b])

    def s_copy(i, b):
      _, _, _, out, orow = chunks[i]
      return pltpu.make_async_copy(
          rows_v.at[b], out.at[pl.ds(orow, CH)], sems_[b])

    for i in range(NB):
      g_copy(i, i).start()
    for i in range(NCH):
      b = i % NB
      g_copy(i, b).wait()
      s_copy(i, b).start()
      if i + NB < NCH:
        s_copy(i, b).wait()
        g_copy(i + NB, b).start()
    for i in range(NCH - NB, NCH):
      s_copy(i, i % NB).wait()

  return k(node_emb_w, a_tab, rel_emb_w, root_idx, rel_idx)


def _tc_dense(g0w, g1w, g2w, W_agg, b_agg, Wk, Wr, b_lstm, W1, b1, W2, b2,
              W3, b3):
  CB = 512
  GRID = B // CB
  bf = jnp.bfloat16

  def unpack(w):
    lo = lax.bitcast_convert_type(w << 16, jnp.float32)
    hi = lax.bitcast_convert_type(w & _HI, jnp.float32)
    return lo, hi

  def body(g00_r, g01_r, g02_r, g10_r, g11_r, g12_r, g20_r, g21_r,
           wagg_r, bagg_r, wk_r, wr_r, bl_r,
           w1_r, b1_r, w2_r, b2_r, w3_r, b3_r, out_r):
    g0_refs = (g00_r, g01_r, g02_r)
    g1_refs = (g10_r, g11_r, g12_r)
    g2_refs = (g20_r, g21_r)
    wagg = wagg_r[...]
    wk = wk_r[...]
    wr = wr_r[...]
    bl = bl_r[...]
    h = None
    c = None
    for s in range(NSTEP):
      lo0, hi0 = unpack(g0_refs[s][...])
      if s > 0:
        lo2, hi2 = unpack(g2_refs[s - 1][...])
        lo0 = lo0 + lo2
        hi0 = hi0 + hi2
      lo1, hi1 = unpack(g1_refs[s][...])
      x = jnp.concatenate([lo0, hi0, lo1, hi1], axis=1)
      feat = jax.nn.relu(
          jnp.dot(x.astype(bf), wagg, preferred_element_type=jnp.float32)
          + bagg_r[...])
      z = jnp.dot(feat.astype(bf), wk, preferred_element_type=jnp.float32) + bl
      if s > 0:
        z = z + jnp.dot(h.astype(bf), wr, preferred_element_type=jnp.float32)
      zi, zf, zg, zo = jnp.split(z, 4, axis=-1)
      ig = jax.nn.sigmoid(zi) * jnp.tanh(zg)
      if s > 0:
        c = jax.nn.sigmoid(zf) * c + ig
      else:
        c = ig
      h = jax.nn.sigmoid(zo) * jnp.tanh(c)
    x = jax.nn.relu(
        jnp.dot(h.astype(bf), w1_r[...], preferred_element_type=jnp.float32)
        + b1_r[...])
    x = jax.nn.relu(
        jnp.dot(x.astype(bf), w2_r[...], preferred_element_type=jnp.float32)
        + b2_r[...])
    lg = (jnp.dot(x.astype(bf), w3_r[...], preferred_element_type=jnp.float32)
          + b3_r[...])
    m = jnp.max(lg, axis=-1, keepdims=True)
    e = jnp.exp(lg - m)
    out_r[...] = e / jnp.sum(e, axis=-1, keepdims=True)

  full = lambda shape: pl.BlockSpec(shape, lambda i: (0,) * len(shape))
  NBLK = B // CB

  def gspec(s):
    return pl.BlockSpec((CB, EW), lambda i, s=s: (s * NBLK + i, 0))

  return pl.pallas_call(
      body,
      grid=(GRID,),
      in_specs=[gspec(0), gspec(1), gspec(2),
                gspec(0), gspec(1), gspec(2),
                gspec(0), gspec(1),
                full((F, F)),
                full((1, F)),
                full((F, 4 * F)),
                full((F, 4 * F)),
                full((1, 4 * F)),
                full((F, H1)),
                full((1, H1)),
                full((H1, H1)),
                full((1, H1)),
                full((H1, 2)),
                full((1, 2))],
      out_specs=pl.BlockSpec((CB, 2), lambda i: (i, 0)),
      out_shape=jax.ShapeDtypeStruct((B, 2), jnp.float32),
      compiler_params=pltpu.CompilerParams(
          dimension_semantics=("arbitrary",)),
  )(g0w, g0w, g0w, g1w, g1w, g1w, g2w, g2w,
    W_agg.astype(bf), b_agg.reshape(1, F), Wk.astype(bf),
    Wr.astype(bf), b_lstm.reshape(1, 4 * F), W1.astype(bf),
    b1.reshape(1, H1), W2.astype(bf), b2.reshape(1, H1), W3.astype(bf),
    b3.reshape(1, 2))


def kernel(paths, node_emb, rel_emb, neighbor_idx, W_agg, b_agg, Wk, Wr,
           b_lstm, W1, b1, W2, b2, W3, b3):
  paths = paths.astype(jnp.int32)
  nidx = jnp.pad(neighbor_idx.astype(jnp.int32), ((0, NPAD - N), (0, 0)))

  def pack_words(x):
    xb = x.astype(jnp.bfloat16).reshape(x.shape[0], EW, 2)
    return lax.bitcast_convert_type(xb, jnp.int32)

  ne_w = pack_words(node_emb)
  rel_w = pack_words(rel_emb)
  a_w = _sc_neighbor_mean(ne_w, nidx.reshape(-1))
  root_idx = paths[:, 0::2].T.reshape(-1)
  rel_idx = paths[:, 1::2].T.reshape(-1)
  g0w, g1w, g2w = _sc_path_gather(ne_w, a_w, rel_w, root_idx, rel_idx)
  ev = jnp.arange(0, E, 2)
  od = ev + 1
  perm = jnp.concatenate([ev, od, E + ev, E + od])
  return _tc_dense(
      g0w, g1w, g2w,
      W_agg[perm], b_agg, Wk, Wr, b_lstm, W1, b1, W2, b2, W3, b3)


# speedup vs baseline: 1.0062x; 1.0062x over previous
import functools

import jax
import jax.numpy as jnp
from jax import lax
from jax.experimental import pallas as pl
from jax.experimental.pallas import tpu as pltpu
from jax.experimental.pallas import tpu_sc as plsc

N = 10000
DEG = 32
E = 128
F = 256
H1 = 400
B = 4096
NSTEP = 3

NC = 2
NS = 16
NW = NC * NS
NPAD = ((N + NW * 8 - 1) // (NW * 8)) * (NW * 8)
NPW = NPAD // NW
RTOT = NSTEP * B
RPW = RTOT // NW
LTOT = (NSTEP - 1) * B
LPW = LTOT // NW
EW = E // 2
NWIN = EW // 16

_mesh = lambda: plsc.VectorSubcoreMesh(core_axis_name="c", subcore_axis_name="s")

_HI = -65536


def _to_f32_pair(w):
  lo = lax.bitcast_convert_type(w << 16, jnp.float32)
  hi = lax.bitcast_convert_type(w & _HI, jnp.float32)
  return lo, hi


def _from_f32_pair(lo, hi):
  bl = lax.shift_right_logical(
      lax.bitcast_convert_type(lo, jnp.int32) + 0x8000, 16)
  bh = (lax.bitcast_convert_type(hi, jnp.int32) + 0x8000) & _HI
  return bl | bh


def _sc_neighbor_mean(emb_w, nidx_flat):
  G4 = 4
  CHI = G4 * DEG
  NGRP = NPW // G4
  NBUF = 4

  SEG = (N + NS - 1) // NS

  @functools.partial(
      pl.kernel,
      out_type=jax.ShapeDtypeStruct((NPAD, EW), jnp.int32),
      mesh=_mesh(),
      scratch_types=[
          pltpu.VMEM((NPW * DEG,), jnp.int32),
          pltpu.VMEM((NBUF, CHI, EW), jnp.int32),
          pltpu.VMEM((NPW, EW), jnp.int32),
          pltpu.VMEM_SHARED((N, EW), jnp.int32),
      ] + [pltpu.SemaphoreType.DMA] * NBUF,
      compiler_params=pltpu.CompilerParams(use_tc_tiling_on_sc=False),
  )
  def k(emb_hbm, idx_hbm, a_hbm, idx_v, rows_v, out_v, emb_sp, *sems):
    wid = lax.axis_index("s") * NC + lax.axis_index("c")
    base = wid * NPW
    sid = lax.axis_index("s")
    seg0 = sid * SEG
    pltpu.sync_copy(idx_hbm.at[pl.ds(base * DEG, NPW * DEG)], idx_v)
    pltpu.sync_copy(emb_hbm.at[pl.ds(seg0, SEG)], emb_sp.at[pl.ds(seg0, SEG)])
    plsc.subcore_barrier()

    def gather(g, b):
      return pltpu.async_copy(
          emb_sp.at[idx_v.at[pl.ds(g * CHI, CHI)]], rows_v.at[b], sems[b])

    for b in range(NBUF):
      gather(b, b)

    def grp_body(it, carry):
      for b in range(NBUF):
        g = it * NBUF + b
        pltpu.make_async_copy(
            emb_sp.at[idx_v.at[pl.ds(g * CHI, CHI)]], rows_v.at[b],
            sems[b]).wait()
        rb = rows_v.at[b]
        for t in range(G4):
          for q in range(NWIN):
            acc_lo = None
            acc_hi = None
            for kk in range(DEG):
              lo, hi = _to_f32_pair(rb[t * DEG + kk, pl.ds(q * 16, 16)])
              acc_lo = lo if acc_lo is None else acc_lo + lo
              acc_hi = hi if acc_hi is None else acc_hi + hi
            out_v[g * G4 + t, pl.ds(q * 16, 16)] = _from_f32_pair(
                acc_lo * (1.0 / DEG), acc_hi * (1.0 / DEG))

        @pl.when(g + NBUF < NGRP)
        def _():
          gather(g + NBUF, b)

      return carry

    lax.fori_loop(0, NGRP // NBUF, grp_body, 0)
    pltpu.sync_copy(out_v, a_hbm.at[pl.ds(base, NPW)])

  return k(emb_w, nidx_flat)


def _sc_path_gather(node_emb_w, a_tab, rel_emb_w, root_idx, rel_idx):
  CH = 128
  NB = 4

  @functools.partial(
      pl.kernel,
      out_type=(
          jax.ShapeDtypeStruct((RTOT, EW), jnp.int32),
          jax.ShapeDtypeStruct((RTOT, EW), jnp.int32),
          jax.ShapeDtypeStruct((LTOT, EW), jnp.int32),
      ),
      mesh=_mesh(),
      scratch_types=[
          pltpu.VMEM((RPW,), jnp.int32),
          pltpu.VMEM((LPW,), jnp.int32),
          pltpu.VMEM((NB, CH, EW), jnp.int32),
      ] + [pltpu.SemaphoreType.DMA] * (2 * NB),
      compiler_params=pltpu.CompilerParams(use_tc_tiling_on_sc=False),
  )
  def k(emb_hbm, a_hbm, rel_hbm, ridx_hbm, lidx_hbm, g0_hbm, g1_hbm, g2_hbm,
        ridx_v, lidx_v, rows_v, *sems):
    semg, sems_ = sems[:NB], sems[NB:]
    wid = lax.axis_index("s") * NC + lax.axis_index("c")
    rbase = wid * RPW
    lbase = wid * LPW
    pltpu.sync_copy(ridx_hbm.at[pl.ds(rbase, RPW)], ridx_v)
    pltpu.sync_copy(lidx_hbm.at[pl.ds(lbase, LPW)], lidx_v)
    chunks = (
        [(emb_hbm, ridx_v, ch * CH, g0_hbm, rbase + ch * CH)
         for ch in range(RPW // CH)]
        + [(a_hbm, ridx_v, ch * CH, g1_hbm, rbase + ch * CH)
           for ch in range(RPW // CH)]
        + [(rel_hbm, lidx_v, ch * CH, g2_hbm, lbase + ch * CH)
           for ch in range(LPW // CH)])
    NCH = len(chunks)

    def g_copy(i, b):
      tab, idxr, ioff, _, _ = chunks[i]
      return pltpu.make_async_copy(
          tab.at[idxr.at[pl.ds(ioff, CH)]], rows_v.at[b], semg[b])

    def s_copy(i, b):
      _, _, _, out, orow = chunks[i]
      return pltpu.make_async_copy(
          rows_v.at[b], out.at[pl.ds(orow, CH)], sems_[b])

    for i in range(NB):
      g_copy(i, i).start()
    for i in range(NCH):
      b = i % NB
      g_copy(i, b).wait()
      s_copy(i, b).start()
      if i + NB < NCH:
        s_copy(i, b).wait()
        g_copy(i + NB, b).start()
    for i in range(NCH - NB, NCH):
      s_copy(i, i % NB).wait()

  return k(node_emb_w, a_tab, rel_emb_w, root_idx, rel_idx)


def _tc_dense(g0w, g1w, g2w, W_agg, b_agg, Wk, Wr, b_lstm, W1, b1, W2, b2,
              W3, b3):
  CB = 1024
  GRID = B // CB
  bf = jnp.bfloat16

  def unpack(w):
    lo = lax.bitcast_convert_type(w << 16, jnp.float32)
    hi = lax.bitcast_convert_type(w & _HI, jnp.float32)
    return lo, hi

  def body(g00_r, g01_r, g02_r, g10_r, g11_r, g12_r, g20_r, g21_r,
           wagg_r, bagg_r, wk_r, wr_r, bl_r,
           w1_r, b1_r, w2_r, b2_r, w3_r, b3_r, out_r):
    g0_refs = (g00_r, g01_r, g02_r)
    g1_refs = (g10_r, g11_r, g12_r)
    g2_refs = (g20_r, g21_r)
    wagg = wagg_r[...]
    wk = wk_r[...]
    wr = wr_r[...]
    bl = bl_r[...]
    h = None
    c = None
    for s in range(NSTEP):
      lo0, hi0 = unpack(g0_refs[s][...])
      if s > 0:
        lo2, hi2 = unpack(g2_refs[s - 1][...])
        lo0 = lo0 + lo2
        hi0 = hi0 + hi2
      lo1, hi1 = unpack(g1_refs[s][...])
      x = jnp.concatenate([lo0, hi0, lo1, hi1], axis=1)
      feat = jax.nn.relu(
          jnp.dot(x.astype(bf), wagg, preferred_element_type=jnp.float32)
          + bagg_r[...])
      z = jnp.dot(feat.astype(bf), wk, preferred_element_type=jnp.float32) + bl
      if s > 0:
        z = z + jnp.dot(h.astype(bf), wr, preferred_element_type=jnp.float32)
      zi, zf, zg, zo = jnp.split(z, 4, axis=-1)
      ig = jax.nn.sigmoid(zi) * jnp.tanh(zg)
      if s > 0:
        c = jax.nn.sigmoid(zf) * c + ig
      else:
        c = ig
      h = jax.nn.sigmoid(zo) * jnp.tanh(c)
    x = jax.nn.relu(
        jnp.dot(h.astype(bf), w1_r[...], preferred_element_type=jnp.float32)
        + b1_r[...])
    x = jax.nn.relu(
        jnp.dot(x.astype(bf), w2_r[...], preferred_element_type=jnp.float32)
        + b2_r[...])
    lg = (jnp.dot(x.astype(bf), w3_r[...], preferred_element_type=jnp.float32)
          + b3_r[...])
    m = jnp.max(lg, axis=-1, keepdims=True)
    e = jnp.exp(lg - m)
    out_r[...] = e / jnp.sum(e, axis=-1, keepdims=True)

  full = lambda shape: pl.BlockSpec(shape, lambda i: (0,) * len(shape))
  NBLK = B // CB

  def gspec(s):
    return pl.BlockSpec((CB, EW), lambda i, s=s: (s * NBLK + i, 0))

  return pl.pallas_call(
      body,
      grid=(GRID,),
      in_specs=[gspec(0), gspec(1), gspec(2),
                gspec(0), gspec(1), gspec(2),
                gspec(0), gspec(1),
                full((F, F)),
                full((1, F)),
                full((F, 4 * F)),
                full((F, 4 * F)),
                full((1, 4 * F)),
                full((F, H1)),
                full((1, H1)),
                full((H1, H1)),
                full((1, H1)),
                full((H1, 2)),
                full((1, 2))],
      out_specs=pl.BlockSpec((CB, 2), lambda i: (i, 0)),
      out_shape=jax.ShapeDtypeStruct((B, 2), jnp.float32),
      compiler_params=pltpu.CompilerParams(
          dimension_semantics=("parallel",)),
  )(g0w, g0w, g0w, g1w, g1w, g1w, g2w, g2w,
    W_agg.astype(bf), b_agg.reshape(1, F), Wk.astype(bf),
    Wr.astype(bf), b_lstm.reshape(1, 4 * F), W1.astype(bf),
    b1.reshape(1, H1), W2.astype(bf), b2.reshape(1, H1), W3.astype(bf),
    b3.reshape(1, 2))


def kernel(paths, node_emb, rel_emb, neighbor_idx, W_agg, b_agg, Wk, Wr,
           b_lstm, W1, b1, W2, b2, W3, b3):
  paths = paths.astype(jnp.int32)
  nidx = jnp.pad(neighbor_idx.astype(jnp.int32), ((0, NPAD - N), (0, 0)))

  def pack_words(x):
    xb = x.astype(jnp.bfloat16).reshape(x.shape[0], EW, 2)
    return lax.bitcast_convert_type(xb, jnp.int32)

  ne_w = pack_words(node_emb)
  rel_w = pack_words(rel_emb)
  a_w = _sc_neighbor_mean(ne_w, nidx.reshape(-1))
  root_idx = paths[:, 0::2].T.reshape(-1)
  rel_idx = paths[:, 1::2].T.reshape(-1)
  g0w, g1w, g2w = _sc_path_gather(ne_w, a_w, rel_w, root_idx, rel_idx)
  ev = jnp.arange(0, E, 2)
  od = ev + 1
  perm = jnp.concatenate([ev, od, E + ev, E + od])
  return _tc_dense(
      g0w, g1w, g2w,
      W_agg[perm], b_agg, Wk, Wr, b_lstm, W1, b1, W2, b2, W3, b3)


# speedup vs baseline: 1.1197x; 1.1128x over previous
import functools

import jax
import jax.numpy as jnp
from jax import lax
from jax.experimental import pallas as pl
from jax.experimental.pallas import tpu as pltpu
from jax.experimental.pallas import tpu_sc as plsc

N = 10000
DEG = 32
E = 128
F = 256
H1 = 400
B = 4096
NSTEP = 3

NC = 2
NS = 16
NW = NC * NS
NPAD = ((N + NW * 8 - 1) // (NW * 8)) * (NW * 8)
NPW = NPAD // NW
RTOT = NSTEP * B
RPW = RTOT // NW
LTOT = (NSTEP - 1) * B
LPW = LTOT // NW
EW = E // 2
NWIN = EW // 16

_mesh = lambda: plsc.VectorSubcoreMesh(core_axis_name="c", subcore_axis_name="s")

_HI = -65536


def _to_f32_pair(w):
  lo = lax.bitcast_convert_type(w << 16, jnp.float32)
  hi = lax.bitcast_convert_type(w & _HI, jnp.float32)
  return lo, hi


def _from_f32_pair(lo, hi):
  bl = lax.shift_right_logical(
      lax.bitcast_convert_type(lo, jnp.int32) + 0x8000, 16)
  bh = (lax.bitcast_convert_type(hi, jnp.int32) + 0x8000) & _HI
  return bl | bh


def _sc_neighbor_mean(emb_w, nidx_flat):
  G4 = 4
  CHI = G4 * DEG
  NGRP = NPW // G4
  NBUF = 2

  SEG = (N + NS - 1) // NS

  @functools.partial(
      pl.kernel,
      out_type=jax.ShapeDtypeStruct((NPAD, EW), jnp.int32),
      mesh=_mesh(),
      scratch_types=[
          pltpu.VMEM((NPW * DEG,), jnp.int32),
          pltpu.VMEM((NBUF, CHI, EW), jnp.int32),
          pltpu.VMEM((NPW, EW), jnp.int32),
          pltpu.VMEM_SHARED((N, EW), jnp.int32),
      ] + [pltpu.SemaphoreType.DMA] * NBUF,
      compiler_params=pltpu.CompilerParams(use_tc_tiling_on_sc=False),
  )
  def k(emb_hbm, idx_hbm, a_hbm, idx_v, rows_v, out_v, emb_sp, *sems):
    wid = lax.axis_index("s") * NC + lax.axis_index("c")
    base = wid * NPW
    sid = lax.axis_index("s")
    seg0 = sid * SEG
    pltpu.sync_copy(idx_hbm.at[pl.ds(base * DEG, NPW * DEG)], idx_v)
    pltpu.sync_copy(emb_hbm.at[pl.ds(seg0, SEG)], emb_sp.at[pl.ds(seg0, SEG)])
    plsc.subcore_barrier()

    def gather(g, b):
      return pltpu.async_copy(
          emb_sp.at[idx_v.at[pl.ds(g * CHI, CHI)]], rows_v.at[b], sems[b])

    for b in range(NBUF):
      gather(b, b)

    def grp_body(it, carry):
      for b in range(NBUF):
        g = it * NBUF + b
        pltpu.make_async_copy(
            emb_sp.at[idx_v.at[pl.ds(g * CHI, CHI)]], rows_v.at[b],
            sems[b]).wait()
        rb = rows_v.at[b]
        for t in range(G4):
          for q in range(NWIN):
            acc_lo = None
            acc_hi = None
            for kk in range(DEG):
              lo, hi = _to_f32_pair(rb[t * DEG + kk, pl.ds(q * 16, 16)])
              acc_lo = lo if acc_lo is None else acc_lo + lo
              acc_hi = hi if acc_hi is None else acc_hi + hi
            out_v[g * G4 + t, pl.ds(q * 16, 16)] = _from_f32_pair(
                acc_lo * (1.0 / DEG), acc_hi * (1.0 / DEG))

        @pl.when(g + NBUF < NGRP)
        def _():
          gather(g + NBUF, b)

      return carry

    lax.fori_loop(0, NGRP // NBUF, grp_body, 0)
    pltpu.sync_copy(out_v, a_hbm.at[pl.ds(base, NPW)])

  return k(emb_w, nidx_flat)


def _sc_path_gather(node_emb_w, a_tab, rel_emb_w, root_idx, rel_idx):
  CH = 128
  NB = 4

  @functools.partial(
      pl.kernel,
      out_type=(
          jax.ShapeDtypeStruct((RTOT, EW), jnp.int32),
          jax.ShapeDtypeStruct((RTOT, EW), jnp.int32),
          jax.ShapeDtypeStruct((LTOT, EW), jnp.int32),
      ),
      mesh=_mesh(),
      scratch_types=[
          pltpu.VMEM((RPW,), jnp.int32),
          pltpu.VMEM((LPW,), jnp.int32),
          pltpu.VMEM((NB, CH, EW), jnp.int32),
      ] + [pltpu.SemaphoreType.DMA] * (2 * NB),
      compiler_params=pltpu.CompilerParams(use_tc_tiling_on_sc=False),
  )
  def k(emb_hbm, a_hbm, rel_hbm, ridx_hbm, lidx_hbm, g0_hbm, g1_hbm, g2_hbm,
        ridx_v, lidx_v, rows_v, *sems):
    semg, sems_ = sems[:NB], sems[NB:]
    wid = lax.axis_index("s") * NC + lax.axis_index("c")
    rbase = wid * RPW
    lbase = wid * LPW
    pltpu.sync_copy(ridx_hbm.at[pl.ds(rbase, RPW)], ridx_v)
    pltpu.sync_copy(lidx_hbm.at[pl.ds(lbase, LPW)], lidx_v)
    chunks = (
        [(emb_hbm, ridx_v, ch * CH, g0_hbm, rbase + ch * CH)
         for ch in range(RPW // CH)]
        + [(a_hbm, ridx_v, ch * CH, g1_hbm, rbase + ch * CH)
           for ch in range(RPW // CH)]
        + [(rel_hbm, lidx_v, ch * CH, g2_hbm, lbase + ch * CH)
           for ch in range(LPW // CH)])
    NCH = len(chunks)

    def g_copy(i, b):
      tab, idxr, ioff, _, _ = chunks[i]
      return pltpu.make_async_copy(
          tab.at[idxr.at[pl.ds(ioff, CH)]], rows_v.at[b], semg[b])

    def s_copy(i, b):
      _, _, _, out, orow = chunks[i]
      return pltpu.make_async_copy(
          rows_v.at[b], out.at[pl.ds(orow, CH)], sems_[b])

    for i in range(NB):
      g_copy(i, i).start()
    for i in range(NCH):
      b = i % NB
      g_copy(i, b).wait()
      s_copy(i, b).start()
      if i + NB < NCH:
        s_copy(i, b).wait()
        g_copy(i + NB, b).start()
    for i in range(NCH - NB, NCH):
      s_copy(i, i % NB).wait()

  return k(node_emb_w, a_tab, rel_emb_w, root_idx, rel_idx)


def _tc_dense(g0w, g1w, g2w, W_agg, b_agg, Wk, Wr, b_lstm, W1, b1, W2, b2,
              W3, b3):
  CB = 1024
  GRID = B // CB
  bf = jnp.bfloat16

  def unpack(w):
    lo = lax.bitcast_convert_type(w << 16, jnp.float32)
    hi = lax.bitcast_convert_type(w & _HI, jnp.float32)
    return lo, hi

  def body(g00_r, g01_r, g02_r, g10_r, g11_r, g12_r, g20_r, g21_r,
           wagg_r, bagg_r, wk_r, wr_r, bl_r,
           w1_r, b1_r, w2_r, b2_r, w3_r, b3_r, out_r):
    g0_refs = (g00_r, g01_r, g02_r)
    g1_refs = (g10_r, g11_r, g12_r)
    g2_refs = (g20_r, g21_r)
    wagg = wagg_r[...]
    wk = wk_r[...]
    wr = wr_r[...]
    bl = bl_r[...]
    h = None
    c = None
    for s in range(NSTEP):
      lo0, hi0 = unpack(g0_refs[s][...])
      if s > 0:
        lo2, hi2 = unpack(g2_refs[s - 1][...])
        lo0 = lo0 + lo2
        hi0 = hi0 + hi2
      lo1, hi1 = unpack(g1_refs[s][...])
      x = jnp.concatenate([lo0, hi0, lo1, hi1], axis=1)
      feat = jax.nn.relu(
          jnp.dot(x.astype(bf), wagg, preferred_element_type=jnp.float32)
          + bagg_r[...])
      z = jnp.dot(feat.astype(bf), wk, preferred_element_type=jnp.float32) + bl
      if s > 0:
        z = z + jnp.dot(h.astype(bf), wr, preferred_element_type=jnp.float32)
      zi, zf, zg, zo = jnp.split(z, 4, axis=-1)
      ig = jax.nn.sigmoid(zi) * jnp.tanh(zg)
      if s > 0:
        c = jax.nn.sigmoid(zf) * c + ig
      else:
        c = ig
      h = jax.nn.sigmoid(zo) * jnp.tanh(c)
    x = jax.nn.relu(
        jnp.dot(h.astype(bf), w1_r[...], preferred_element_type=jnp.float32)
        + b1_r[...])
    x = jax.nn.relu(
        jnp.dot(x.astype(bf), w2_r[...], preferred_element_type=jnp.float32)
        + b2_r[...])
    lg = (jnp.dot(x.astype(bf), w3_r[...], preferred_element_type=jnp.float32)
          + b3_r[...])
    m = jnp.max(lg, axis=-1, keepdims=True)
    e = jnp.exp(lg - m)
    out_r[...] = e / jnp.sum(e, axis=-1, keepdims=True)

  full = lambda shape: pl.BlockSpec(shape, lambda i: (0,) * len(shape))
  NBLK = B // CB

  def gspec(s):
    return pl.BlockSpec((CB, EW), lambda i, s=s: (s * NBLK + i, 0))

  return pl.pallas_call(
      body,
      grid=(GRID,),
      in_specs=[gspec(0), gspec(1), gspec(2),
                gspec(0), gspec(1), gspec(2),
                gspec(0), gspec(1),
                full((F, F)),
                full((1, F)),
                full((F, 4 * F)),
                full((F, 4 * F)),
                full((1, 4 * F)),
                full((F, H1)),
                full((1, H1)),
                full((H1, H1)),
                full((1, H1)),
                full((H1, 2)),
                full((1, 2))],
      out_specs=pl.BlockSpec((CB, 2), lambda i: (i, 0)),
      out_shape=jax.ShapeDtypeStruct((B, 2), jnp.float32),
      compiler_params=pltpu.CompilerParams(
          dimension_semantics=("parallel",)),
  )(g0w, g0w, g0w, g1w, g1w, g1w, g2w, g2w,
    W_agg.astype(bf), b_agg.reshape(1, F), Wk.astype(bf),
    Wr.astype(bf), b_lstm.reshape(1, 4 * F), W1.astype(bf),
    b1.reshape(1, H1), W2.astype(bf), b2.reshape(1, H1), W3.astype(bf),
    b3.reshape(1, 2))


def kernel(paths, node_emb, rel_emb, neighbor_idx, W_agg, b_agg, Wk, Wr,
           b_lstm, W1, b1, W2, b2, W3, b3):
  paths = paths.astype(jnp.int32)
  nidx = jnp.pad(neighbor_idx.astype(jnp.int32), ((0, NPAD - N), (0, 0)))

  def pack_words(x):
    xb = x.astype(jnp.bfloat16).reshape(x.shape[0], EW, 2)
    return lax.bitcast_convert_type(xb, jnp.int32)

  ne_w = pack_words(node_emb)
  rel_w = pack_words(rel_emb)
  a_w = _sc_neighbor_mean(ne_w, nidx.reshape(-1))
  root_idx = paths[:, 0::2].T.reshape(-1)
  rel_idx = paths[:, 1::2].T.reshape(-1)
  g0w, g1w, g2w = _sc_path_gather(ne_w, a_w, rel_w, root_idx, rel_idx)
  ev = jnp.arange(0, E, 2)
  od = ev + 1
  perm = jnp.concatenate([ev, od, E + ev, E + od])
  return _tc_dense(
      g0w, g1w, g2w,
      W_agg[perm], b_agg, Wk, Wr, b_lstm, W1, b1, W2, b2, W3, b3)


# speedup vs baseline: 1.1239x; 1.0037x over previous
import functools

import jax
import jax.numpy as jnp
from jax import lax
from jax.experimental import pallas as pl
from jax.experimental.pallas import tpu as pltpu
from jax.experimental.pallas import tpu_sc as plsc

N = 10000
DEG = 32
E = 128
F = 256
H1 = 400
B = 4096
NSTEP = 3

NC = 2
NS = 16
NW = NC * NS
NPAD = ((N + NW * 8 - 1) // (NW * 8)) * (NW * 8)
NPW = NPAD // NW
RTOT = NSTEP * B
RPW = RTOT // NW
LTOT = (NSTEP - 1) * B
LPW = LTOT // NW
EW = E // 2
NWIN = EW // 16

_mesh = lambda: plsc.VectorSubcoreMesh(core_axis_name="c", subcore_axis_name="s")

_HI = -65536


def _to_f32_pair(w):
  lo = lax.bitcast_convert_type(w << 16, jnp.float32)
  hi = lax.bitcast_convert_type(w & _HI, jnp.float32)
  return lo, hi


def _from_f32_pair(lo, hi):
  bl = lax.shift_right_logical(
      lax.bitcast_convert_type(lo, jnp.int32) + 0x8000, 16)
  bh = (lax.bitcast_convert_type(hi, jnp.int32) + 0x8000) & _HI
  return bl | bh


def _sc_neighbor_mean(emb_w, nidx_flat):
  G4 = 4
  CHI = G4 * DEG
  NGRP = NPW // G4
  NBUF = 2

  SEG = (N + NS - 1) // NS

  @functools.partial(
      pl.kernel,
      out_type=jax.ShapeDtypeStruct((N, EW), jnp.int32),
      mesh=_mesh(),
      scratch_types=[
          pltpu.VMEM((NPW * DEG,), jnp.int32),
          pltpu.VMEM((NBUF, CHI, EW), jnp.int32),
          pltpu.VMEM((NPW, EW), jnp.int32),
          pltpu.VMEM_SHARED((N, EW), jnp.int32),
      ] + [pltpu.SemaphoreType.DMA] * NBUF,
      compiler_params=pltpu.CompilerParams(use_tc_tiling_on_sc=False),
  )
  def k(emb_hbm, idx_hbm, a_hbm, idx_v, rows_v, out_v, emb_sp, *sems):
    wid = lax.axis_index("s") * NC + lax.axis_index("c")
    base = jnp.minimum(wid * NPW, N - NPW)
    sid = lax.axis_index("s")
    seg0 = sid * SEG
    pltpu.sync_copy(idx_hbm.at[pl.ds(base * DEG, NPW * DEG)], idx_v)
    pltpu.sync_copy(emb_hbm.at[pl.ds(seg0, SEG)], emb_sp.at[pl.ds(seg0, SEG)])
    plsc.subcore_barrier()

    def gather(g, b):
      return pltpu.async_copy(
          emb_sp.at[idx_v.at[pl.ds(g * CHI, CHI)]], rows_v.at[b], sems[b])

    for b in range(NBUF):
      gather(b, b)

    def grp_body(it, carry):
      for b in range(NBUF):
        g = it * NBUF + b
        pltpu.make_async_copy(
            emb_sp.at[idx_v.at[pl.ds(g * CHI, CHI)]], rows_v.at[b],
            sems[b]).wait()
        rb = rows_v.at[b]
        for t in range(G4):
          for q in range(NWIN):
            acc_lo = None
            acc_hi = None
            for kk in range(DEG):
              lo, hi = _to_f32_pair(rb[t * DEG + kk, pl.ds(q * 16, 16)])
              acc_lo = lo if acc_lo is None else acc_lo + lo
              acc_hi = hi if acc_hi is None else acc_hi + hi
            out_v[g * G4 + t, pl.ds(q * 16, 16)] = _from_f32_pair(
                acc_lo * (1.0 / DEG), acc_hi * (1.0 / DEG))

        @pl.when(g + NBUF < NGRP)
        def _():
          gather(g + NBUF, b)

      return carry

    lax.fori_loop(0, NGRP // NBUF, grp_body, 0)
    pltpu.sync_copy(out_v, a_hbm.at[pl.ds(base, NPW)])

  return k(emb_w, nidx_flat)


def _sc_path_gather(node_emb_w, a_tab, rel_emb_w, root_idx, rel_idx):
  CH = 128
  NB = 4

  @functools.partial(
      pl.kernel,
      out_type=(
          jax.ShapeDtypeStruct((RTOT, EW), jnp.int32),
          jax.ShapeDtypeStruct((RTOT, EW), jnp.int32),
          jax.ShapeDtypeStruct((LTOT, EW), jnp.int32),
      ),
      mesh=_mesh(),
      scratch_types=[
          pltpu.VMEM((RPW,), jnp.int32),
          pltpu.VMEM((LPW,), jnp.int32),
          pltpu.VMEM((NB, CH, EW), jnp.int32),
      ] + [pltpu.SemaphoreType.DMA] * (2 * NB),
      compiler_params=pltpu.CompilerParams(use_tc_tiling_on_sc=False),
  )
  def k(emb_hbm, a_hbm, rel_hbm, ridx_hbm, lidx_hbm, g0_hbm, g1_hbm, g2_hbm,
        ridx_v, lidx_v, rows_v, *sems):
    semg, sems_ = sems[:NB], sems[NB:]
    wid = lax.axis_index("s") * NC + lax.axis_index("c")
    rbase = wid * RPW
    lbase = wid * LPW
    pltpu.sync_copy(ridx_hbm.at[pl.ds(rbase, RPW)], ridx_v)
    pltpu.sync_copy(lidx_hbm.at[pl.ds(lbase, LPW)], lidx_v)
    chunks = (
        [(emb_hbm, ridx_v, ch * CH, g0_hbm, rbase + ch * CH)
         for ch in range(RPW // CH)]
        + [(a_hbm, ridx_v, ch * CH, g1_hbm, rbase + ch * CH)
           for ch in range(RPW // CH)]
        + [(rel_hbm, lidx_v, ch * CH, g2_hbm, lbase + ch * CH)
           for ch in range(LPW // CH)])
    NCH = len(chunks)

    def g_copy(i, b):
      tab, idxr, ioff, _, _ = chunks[i]
      return pltpu.make_async_copy(
          tab.at[idxr.at[pl.ds(ioff, CH)]], rows_v.at[b], semg[b])

    def s_copy(i, b):
      _, _, _, out, orow = chunks[i]
      return pltpu.make_async_copy(
          rows_v.at[b], out.at[pl.ds(orow, CH)], sems_[b])

    for i in range(NB):
      g_copy(i, i).start()
    for i in range(NCH):
      b = i % NB
      g_copy(i, b).wait()
      s_copy(i, b).start()
      if i + NB < NCH:
        s_copy(i, b).wait()
        g_copy(i + NB, b).start()
    for i in range(NCH - NB, NCH):
      s_copy(i, i % NB).wait()

  return k(node_emb_w, a_tab, rel_emb_w, root_idx, rel_idx)


def _tc_dense(g0w, g1w, g2w, W_agg, b_agg, Wk, Wr, b_lstm, W1, b1, W2, b2,
              W3, b3):
  CB = 1024
  GRID = B // CB
  bf = jnp.bfloat16

  def unpack(w):
    lo = lax.bitcast_convert_type(w << 16, jnp.float32)
    hi = lax.bitcast_convert_type(w & _HI, jnp.float32)
    return lo, hi

  def body(g00_r, g01_r, g02_r, g10_r, g11_r, g12_r, g20_r, g21_r,
           wagg_r, bagg_r, wk_r, wr_r, bl_r,
           w1_r, b1_r, w2_r, b2_r, w3_r, b3_r, out_r):
    g0_refs = (g00_r, g01_r, g02_r)
    g1_refs = (g10_r, g11_r, g12_r)
    g2_refs = (g20_r, g21_r)
    wagg = wagg_r[...]
    wk = wk_r[...]
    wr = wr_r[...]
    bl = bl_r[...]
    h = None
    c = None
    for s in range(NSTEP):
      lo0, hi0 = unpack(g0_refs[s][...])
      if s > 0:
        lo2, hi2 = unpack(g2_refs[s - 1][...])
        lo0 = lo0 + lo2
        hi0 = hi0 + hi2
      lo1, hi1 = unpack(g1_refs[s][...])
      x = jnp.concatenate([lo0, hi0, lo1, hi1], axis=1)
      feat = jax.nn.relu(
          jnp.dot(x.astype(bf), wagg, preferred_element_type=jnp.float32)
          + bagg_r[...])
      z = jnp.dot(feat.astype(bf), wk, preferred_element_type=jnp.float32) + bl
      if s > 0:
        z = z + jnp.dot(h.astype(bf), wr, preferred_element_type=jnp.float32)
      zi, zf, zg, zo = jnp.split(z, 4, axis=-1)
      ig = jax.nn.sigmoid(zi) * jnp.tanh(zg)
      if s > 0:
        c = jax.nn.sigmoid(zf) * c + ig
      else:
        c = ig
      h = jax.nn.sigmoid(zo) * jnp.tanh(c)
    x = jax.nn.relu(
        jnp.dot(h.astype(bf), w1_r[...], preferred_element_type=jnp.float32)
        + b1_r[...])
    x = jax.nn.relu(
        jnp.dot(x.astype(bf), w2_r[...], preferred_element_type=jnp.float32)
        + b2_r[...])
    lg = (jnp.dot(x.astype(bf), w3_r[...], preferred_element_type=jnp.float32)
          + b3_r[...])
    m = jnp.max(lg, axis=-1, keepdims=True)
    e = jnp.exp(lg - m)
    out_r[...] = e / jnp.sum(e, axis=-1, keepdims=True)

  full = lambda shape: pl.BlockSpec(shape, lambda i: (0,) * len(shape))
  NBLK = B // CB

  def gspec(s):
    return pl.BlockSpec((CB, EW), lambda i, s=s: (s * NBLK + i, 0))

  return pl.pallas_call(
      body,
      grid=(GRID,),
      in_specs=[gspec(0), gspec(1), gspec(2),
                gspec(0), gspec(1), gspec(2),
                gspec(0), gspec(1),
                full((F, F)),
                full((1, F)),
                full((F, 4 * F)),
                full((F, 4 * F)),
                full((1, 4 * F)),
                full((F, H1)),
                full((1, H1)),
                full((H1, H1)),
                full((1, H1)),
                full((H1, 2)),
                full((1, 2))],
      out_specs=pl.BlockSpec((CB, 2), lambda i: (i, 0)),
      out_shape=jax.ShapeDtypeStruct((B, 2), jnp.float32),
      compiler_params=pltpu.CompilerParams(
          dimension_semantics=("parallel",)),
  )(g0w, g0w, g0w, g1w, g1w, g1w, g2w, g2w,
    W_agg.astype(bf), b_agg.reshape(1, F), Wk.astype(bf),
    Wr.astype(bf), b_lstm.reshape(1, 4 * F), W1.astype(bf),
    b1.reshape(1, H1), W2.astype(bf), b2.reshape(1, H1), W3.astype(bf),
    b3.reshape(1, 2))


def kernel(paths, node_emb, rel_emb, neighbor_idx, W_agg, b_agg, Wk, Wr,
           b_lstm, W1, b1, W2, b2, W3, b3):
  paths = paths.astype(jnp.int32)
  nidx = neighbor_idx.astype(jnp.int32)

  def pack_words(x):
    xb = x.astype(jnp.bfloat16).reshape(x.shape[0], EW, 2)
    return lax.bitcast_convert_type(xb, jnp.int32)

  ne_w = pack_words(node_emb)
  rel_w = pack_words(rel_emb)
  a_w = _sc_neighbor_mean(ne_w, nidx.reshape(-1))
  root_idx = paths[:, 0::2].T.reshape(-1)
  rel_idx = paths[:, 1::2].T.reshape(-1)
  g0w, g1w, g2w = _sc_path_gather(ne_w, a_w, rel_w, root_idx, rel_idx)
  ev = jnp.arange(0, E, 2)
  od = ev + 1
  perm = jnp.concatenate([ev, od, E + ev, E + od])
  return _tc_dense(
      g0w, g1w, g2w,
      W_agg[perm], b_agg, Wk, Wr, b_lstm, W1, b1, W2, b2, W3, b3)
